# baseline (speedup 1.0000x reference)
import functools

import jax
import jax.numpy as jnp
from jax import lax
from jax.experimental import pallas as pl
from jax.experimental.pallas import tpu as pltpu
from jax.experimental.pallas import tpu_sc as plsc

_N_CELLTYPES = 10
_N_CELLS = 1000
_N_GENES = 5000
_TOTAL = 500
_LANES = 16


def _in_graph_perms():
    key = jax.random.key(42)
    ks = jax.random.split(key, 2)
    kperm = jax.random.split(ks[-1], _N_CELLTYPES)
    return jnp.stack(
        [jax.random.permutation(kperm[c], _N_CELLS) for c in range(_N_CELLTYPES)]
    ).astype(jnp.int32)


def _perm_table():
    import numpy as np

    try:
        with jax.default_device(jax.devices("cpu")[0]):
            return np.asarray(_in_graph_perms()).astype(np.int32)
    except Exception:
        return None


_PERMS = _perm_table()


def _sample_counts(alpha):
    key = jax.random.key(42)
    ks = jax.random.split(key, alpha.shape[0] + 1)

    def dm(k, a):
        kd, kc = jax.random.split(k)
        p = jax.random.dirichlet(kd, a)
        draws = jax.random.categorical(kc, jnp.log(p), shape=(_TOTAL,))
        cats = jnp.arange(a.shape[-1], dtype=draws.dtype)
        return jnp.sum((draws[:, None] == cats[None, :]).astype(jnp.int32), axis=0)

    counts = jax.vmap(dm)(ks[:-1], alpha).astype(jnp.int32)
    perms = _in_graph_perms() if _PERMS is None else jnp.asarray(_PERMS)
    return counts[0], perms


def _row_ids(c0, perms, n_pad_to):
    cs = jnp.cumsum(c0)
    j = jnp.arange(n_pad_to, dtype=jnp.int32)
    ge = j[:, None] >= cs[None, :]
    c_of = jnp.minimum(jnp.sum(ge, axis=1), _N_CELLTYPES - 1).astype(jnp.int32)
    i = j - jnp.sum(ge * c0[None, :], axis=1).astype(jnp.int32)
    flat = c_of * _N_CELLS + jnp.clip(i, 0, _N_CELLS - 1)
    rows = c_of * _N_CELLS + perms.reshape(-1)[flat]
    return jnp.where(j < _TOTAL, rows, 0).astype(jnp.int32)


def _sc_gather_sum(scdata2d, idx, n_workers, rows_per_worker):
    g = _N_GENES
    g_full = (g // _LANES) * _LANES
    tail_off = g - _LANES
    last_valid = _TOTAL - (n_workers - 1) * rows_per_worker
    mesh = plsc.VectorSubcoreMesh(core_axis_name="c", subcore_axis_name="s")
    num_cores = n_workers // 16

    @functools.partial(
        pl.kernel,
        mesh=mesh,
        out_type=jax.ShapeDtypeStruct((n_workers, g), jnp.float32),
        scratch_types=[
            pltpu.VMEM((rows_per_worker,), jnp.int32),
            pltpu.VMEM((rows_per_worker, g), jnp.float32),
            pltpu.VMEM((g,), jnp.float32),
            pltpu.SemaphoreType.DMA,
        ],
    )
    def k(scdata_hbm, idx_hbm, out_hbm, idx_v, rows_v, acc_v, sem):
        wid = lax.axis_index("s") * num_cores + lax.axis_index("c")
        base = wid * rows_per_worker
        pltpu.sync_copy(idx_hbm.at[pl.ds(base, rows_per_worker)], idx_v)
        iv = idx_v[...]

        def accum(nrows):
            copies = [
                pltpu.async_copy(scdata_hbm.at[iv[j]], rows_v.at[j], sem)
                for j in range(nrows)
            ]
            half = min(8, nrows)
            for c in copies[:half]:
                c.wait()

            def chunk_a(ci, carry):
                off = pl.multiple_of(ci * _LANES, _LANES)
                s = rows_v[0, pl.ds(off, _LANES)]
                for j in range(1, half):
                    s = s + rows_v[j, pl.ds(off, _LANES)]
                acc_v[pl.ds(off, _LANES)] = s
                return carry

            lax.fori_loop(0, g_full // _LANES, chunk_a, 0)
            if nrows > half:
                for c in copies[half:]:
                    c.wait()

                def chunk_b(ci, carry):
                    off = pl.multiple_of(ci * _LANES, _LANES)
                    s = rows_v[half, pl.ds(off, _LANES)]
                    for j in range(half + 1, nrows):
                        s = s + rows_v[j, pl.ds(off, _LANES)]
                    acc_v[pl.ds(off, _LANES)] = acc_v[pl.ds(off, _LANES)] + s
                    return carry

                lax.fori_loop(0, g_full // _LANES, chunk_b, 0)
            t = acc_v[pl.ds(tail_off, _LANES)]
            s = rows_v[0, pl.ds(tail_off, _LANES)]
            for j in range(1, nrows):
                s = s + rows_v[j, pl.ds(tail_off, _LANES)]
            lane = lax.iota(jnp.int32, _LANES)
            acc_v[pl.ds(tail_off, _LANES)] = jnp.where(
                lane < (g_full - tail_off), t, s
            )

        @pl.when(wid < n_workers - 1)
        def _():
            accum(rows_per_worker)

        @pl.when(wid == n_workers - 1)
        def _():
            accum(last_valid)

        pltpu.sync_copy(acc_v, out_hbm.at[wid])

    return k(scdata2d, idx)


def _post_body(part_ref, g_ref, b_ref, o_ref):
    acc = jnp.sum(part_ref[...], axis=0, keepdims=True)
    y = jnp.log1p(acc)
    mean = jnp.mean(y)
    var = jnp.mean((y - mean) ** 2)
    yn = (y - mean) / jnp.sqrt(var + 1e-3) * g_ref[...] + b_ref[...]
    mn = jnp.min(yn)
    mx = jnp.max(yn)
    num = yn - mn
    den = mx - mn
    o_ref[...] = jnp.where(den == 0, jnp.zeros_like(num), num / den)


def kernel(x, scdata, W, b, gamma, beta):
    h = jax.nn.relu(x @ W + b)
    alpha = jnp.maximum(h, 1e-6)
    c0, perms = _sample_counts(alpha)

    n_workers = 32
    rows_per_worker = 16
    idx = _row_ids(c0, perms, n_workers * rows_per_worker)

    scdata2d = scdata.reshape(_N_CELLTYPES * _N_CELLS, _N_GENES)
    partials = _sc_gather_sum(scdata2d, idx, n_workers, rows_per_worker)

    out = pl.pallas_call(
        _post_body,
        out_shape=jax.ShapeDtypeStruct((1, _N_GENES), jnp.float32),
    )(partials, gamma.reshape(1, _N_GENES), beta.reshape(1, _N_GENES))
    return out.reshape(_N_GENES)


# speedup vs baseline: 1.0040x; 1.0040x over previous
import functools

import jax
import jax.numpy as jnp
from jax import lax
from jax.experimental import pallas as pl
from jax.experimental.pallas import tpu as pltpu
from jax.experimental.pallas import tpu_sc as plsc

_N_CELLTYPES = 10
_N_CELLS = 1000
_N_GENES = 5000
_TOTAL = 500
_LANES = 16


def _in_graph_perms():
    key = jax.random.key(42)
    ks = jax.random.split(key, 2)
    kperm = jax.random.split(ks[-1], _N_CELLTYPES)
    return jnp.stack(
        [jax.random.permutation(kperm[c], _N_CELLS) for c in range(_N_CELLTYPES)]
    ).astype(jnp.int32)


def _perm_table():
    import numpy as np

    try:
        with jax.default_device(jax.devices("cpu")[0]):
            return np.asarray(_in_graph_perms()).astype(np.int32)
    except Exception:
        return None


_PERMS = _perm_table()


def _sample_counts(alpha):
    key = jax.random.key(42)
    ks = jax.random.split(key, alpha.shape[0] + 1)

    def dm(k, a):
        kd, kc = jax.random.split(k)
        p = jax.random.dirichlet(kd, a)
        draws = jax.random.categorical(kc, jnp.log(p), shape=(_TOTAL,))
        cats = jnp.arange(a.shape[-1], dtype=draws.dtype)
        return jnp.sum((draws[:, None] == cats[None, :]).astype(jnp.int32), axis=0)

    counts = jax.vmap(dm)(ks[:-1], alpha).astype(jnp.int32)
    perms = _in_graph_perms() if _PERMS is None else jnp.asarray(_PERMS)
    return counts[0], perms


def _row_ids(c0, perms, n_pad_to):
    cs = jnp.cumsum(c0)
    j = jnp.arange(n_pad_to, dtype=jnp.int32)
    ge = j[:, None] >= cs[None, :]
    c_of = jnp.minimum(jnp.sum(ge, axis=1), _N_CELLTYPES - 1).astype(jnp.int32)
    i = j - jnp.sum(ge * c0[None, :], axis=1).astype(jnp.int32)
    flat = c_of * _N_CELLS + jnp.clip(i, 0, _N_CELLS - 1)
    rows = c_of * _N_CELLS + perms.reshape(-1)[flat]
    return jnp.where(j < _TOTAL, rows, 0).astype(jnp.int32)


def _sc_gather_sum(scdata2d, idx, n_workers, rows_per_worker):
    g = _N_GENES
    g_full = (g // _LANES) * _LANES
    tail_off = g - _LANES
    last_valid = _TOTAL - (n_workers - 1) * rows_per_worker
    mesh = plsc.VectorSubcoreMesh(core_axis_name="c", subcore_axis_name="s")
    num_cores = n_workers // 16

    @functools.partial(
        pl.kernel,
        mesh=mesh,
        out_type=jax.ShapeDtypeStruct((n_workers, g), jnp.float32),
        scratch_types=[
            pltpu.VMEM((rows_per_worker,), jnp.int32),
            pltpu.VMEM((rows_per_worker, g), jnp.float32),
            pltpu.VMEM((g,), jnp.float32),
            pltpu.SemaphoreType.DMA,
        ],
    )
    def k(scdata_hbm, idx_hbm, out_hbm, idx_v, rows_v, acc_v, sem):
        wid = lax.axis_index("s") * num_cores + lax.axis_index("c")
        base = wid * rows_per_worker
        pltpu.sync_copy(idx_hbm.at[pl.ds(base, rows_per_worker)], idx_v)
        iv = idx_v[...]

        def accum(nrows):
            copies = [
                pltpu.async_copy(scdata_hbm.at[iv[j]], rows_v.at[j], sem)
                for j in range(nrows)
            ]
            for c in copies:
                c.wait()

            def chunk(ci, carry):
                off = pl.multiple_of(ci * _LANES, _LANES)
                s = rows_v[0, pl.ds(off, _LANES)]
                for j in range(1, nrows):
                    s = s + rows_v[j, pl.ds(off, _LANES)]
                acc_v[pl.ds(off, _LANES)] = s
                return carry

            lax.fori_loop(0, g_full // _LANES, chunk, 0)
            t = acc_v[pl.ds(tail_off, _LANES)]
            s = rows_v[0, pl.ds(tail_off, _LANES)]
            for j in range(1, nrows):
                s = s + rows_v[j, pl.ds(tail_off, _LANES)]
            lane = lax.iota(jnp.int32, _LANES)
            acc_v[pl.ds(tail_off, _LANES)] = jnp.where(
                lane < (g_full - tail_off), t, s
            )

        @pl.when(wid < n_workers - 1)
        def _():
            accum(rows_per_worker)

        @pl.when(wid == n_workers - 1)
        def _():
            accum(last_valid)

        pltpu.sync_copy(acc_v, out_hbm.at[wid])

    return k(scdata2d, idx)


def _post_body(part_ref, g_ref, b_ref, o_ref):
    acc = jnp.sum(part_ref[...], axis=0, keepdims=True)
    y = jnp.log1p(acc)
    mean = jnp.mean(y)
    var = jnp.mean((y - mean) ** 2)
    yn = (y - mean) / jnp.sqrt(var + 1e-3) * g_ref[...] + b_ref[...]
    mn = jnp.min(yn)
    mx = jnp.max(yn)
    num = yn - mn
    den = mx - mn
    o_ref[...] = jnp.where(den == 0, jnp.zeros_like(num), num / den)


def kernel(x, scdata, W, b, gamma, beta):
    h = jax.nn.relu(x @ W + b)
    alpha = jnp.maximum(h, 1e-6)
    c0, perms = _sample_counts(alpha)

    n_workers = 32
    rows_per_worker = 16
    idx = _row_ids(c0, perms, n_workers * rows_per_worker)

    scdata2d = scdata.reshape(_N_CELLTYPES * _N_CELLS, _N_GENES)
    partials = _sc_gather_sum(scdata2d, idx, n_workers, rows_per_worker)

    out = pl.pallas_call(
        _post_body,
        out_shape=jax.ShapeDtypeStruct((1, _N_GENES), jnp.float32),
    )(partials, gamma.reshape(1, _N_GENES), beta.reshape(1, _N_GENES))
    return out.reshape(_N_GENES)
